# baseline (speedup 1.0000x reference)
import functools

import jax
import jax.numpy as jnp
from jax import lax
from jax.experimental import pallas as pl
from jax.experimental.pallas import tpu as pltpu
from jax.experimental.pallas import tpu_sc as plsc

N = 10000
E = 320000
F = 128
C = 128
CI = 256
DS = 16
DC = 4
DTR = 8
OUT = 128

L = 125
NCH = 80

NW = 32
KE = 128
CH0 = 60
CH1 = 100
TOTCH = 16 * (CH0 + CH1)
EPAD = TOTCH * KE
NROWS = 10112


def _silu(a):
    return a * (1.0 / (1.0 + jnp.exp(-a)))


def _softplus(a):
    return jnp.log(1.0 + jnp.exp(-jnp.abs(a))) + jnp.maximum(a, 0.0)


def _pre_body(x_ref, wemb_ref, bemb_ref, winp_ref, binp_ref, convt_ref,
              bconv_ref, wxp_ref, wdt_ref, bdt_ref,
              dt_ref, u_ref, z_ref, bc_ref, halo_ref):
    c = pl.program_id(0)
    xblk = x_ref[0]
    h = jnp.dot(xblk, wemb_ref[...], preferred_element_type=jnp.float32)
    h = h + bemb_ref[...]
    xz = jnp.dot(h, winp_ref[...], preferred_element_type=jnp.float32)
    xz = xz + binp_ref[...]
    xm = xz[:, :CI]
    z = xz[:, CI:]

    @pl.when(c == 0)
    def _():
        halo_ref[...] = jnp.zeros_like(halo_ref)

    halo = halo_ref[...]
    xm_ext = jnp.concatenate([halo[5:8], xm], axis=0)
    halo_ref[5:8] = xm[L - 3:L]

    xc = (xm_ext[0:L] * convt_ref[0:1] + xm_ext[1:L + 1] * convt_ref[1:2]
          + xm_ext[2:L + 2] * convt_ref[2:3] + xm_ext[3:L + 3] * convt_ref[3:4])
    xm2 = _silu(xc + bconv_ref[...])
    x_dbl = jnp.dot(xm2, wxp_ref[...], preferred_element_type=jnp.float32)
    dt = _softplus(jnp.dot(x_dbl[:, :DTR], wdt_ref[...],
                           preferred_element_type=jnp.float32) + bdt_ref[...])

    dt_ref[...] = dt[:, None, None, :]
    u_ref[...] = xm2[:, None, None, :]
    z_ref[...] = z[:, None, None, :]
    bc_ref[...] = x_dbl[:, None, None, DTR:DTR + 2 * DS]


def _pre(x, W_emb, b_emb, in_proj_W, in_proj_b, conv_Wt, conv_b,
         x_proj_W, dt_proj_W, dt_proj_b):
    full = lambda shape: pl.BlockSpec(shape, lambda c: (0,) * len(shape))
    return pl.pallas_call(
        _pre_body,
        grid=(NCH,),
        in_specs=[
            pl.BlockSpec((1, L, F), lambda c: (c, 0, 0)),
            full((F, C)), full((1, C)), full((C, 2 * CI)), full((1, 2 * CI)),
            full((DC, CI)), full((1, CI)), full((CI, DTR + 2 * DS)),
            full((DTR, CI)), full((1, CI)),
        ],
        out_specs=[
            pl.BlockSpec((L, 1, 1, CI), lambda c: (0, c, 0, 0)),
            pl.BlockSpec((L, 1, 1, CI), lambda c: (0, c, 0, 0)),
            pl.BlockSpec((L, 1, 1, CI), lambda c: (0, c, 0, 0)),
            pl.BlockSpec((L, 1, 1, 2 * DS), lambda c: (0, c, 0, 0)),
        ],
        out_shape=[
            jax.ShapeDtypeStruct((L, NCH, 1, CI), jnp.float32),
            jax.ShapeDtypeStruct((L, NCH, 1, CI), jnp.float32),
            jax.ShapeDtypeStruct((L, NCH, 1, CI), jnp.float32),
            jax.ShapeDtypeStruct((L, NCH, 1, 2 * DS), jnp.float32),
        ],
        scratch_shapes=[pltpu.VMEM((8, CI), jnp.float32)],
    )(x.reshape(NCH, L, F), W_emb, b_emb, in_proj_W, in_proj_b, conv_Wt,
      conv_b, x_proj_W, dt_proj_W, dt_proj_b)


def _scan_body(dt_ref, u_ref, z_ref, bc_ref, alogt_ref, d_ref,
               wout_ref, bout_ref, wproj_ref, bproj_ref, wmsg_ref, bmsg_ref,
               p_ref, q_ref, s_ref, i_ref, g_ref):
    A_T = -jnp.exp(alogt_ref[...])

    s_ref[...] = jnp.zeros_like(s_ref)

    def pass1(t, g):
        dt_t = dt_ref[t]
        u_t = u_ref[t]
        b_t = bc_ref[t][:, :DS]
        dA = jnp.exp(dt_t[:, None, :] * A_T[None])
        s_ref[...] = s_ref[...] * dA + (dt_t * u_t)[:, None, :] * b_t[:, :, None]
        return g + dt_t

    g_ref[...] = lax.fori_loop(0, L, pass1, jnp.zeros((NCH, CI), jnp.float32))

    def comb(c, S):
        i_ref[c] = S
        decay = jnp.exp(g_ref[c][None, :] * A_T)
        return s_ref[c] + decay * S

    lax.fori_loop(0, NCH, comb, jnp.zeros((DS, CI), jnp.float32))

    def pass2(t, _):
        dt_t = dt_ref[t]
        u_t = u_ref[t]
        bc_t = bc_ref[t]
        b_t = bc_t[:, :DS]
        c_t = bc_t[:, DS:]
        dA = jnp.exp(dt_t[:, None, :] * A_T[None])
        s = i_ref[...] * dA + (dt_t * u_t)[:, None, :] * b_t[:, :, None]
        i_ref[...] = s
        y = jnp.sum(s * c_t[:, :, None], axis=1) + d_ref[...] * u_t
        y = y * _silu(z_ref[t])
        h2 = jnp.dot(y, wout_ref[...], preferred_element_type=jnp.float32)
        h2 = h2 + bout_ref[...]
        p = jnp.dot(h2, wproj_ref[...], preferred_element_type=jnp.float32)
        p = p + bproj_ref[...]
        q = _silu(jnp.dot(p, wmsg_ref[...], preferred_element_type=jnp.float32)
                  + bmsg_ref[...])
        p_ref[:, t, :] = p
        q_ref[:, t, :] = q
        return 0

    lax.fori_loop(0, L, pass2, 0)


def _scan(dt, u, z, bc, A_logT, D_skip, out_proj_W, out_proj_b,
          W_proj, b_proj, W_msg, b_msg):
    return pl.pallas_call(
        _scan_body,
        out_shape=[
            jax.ShapeDtypeStruct((NCH, L, F), jnp.float32),
            jax.ShapeDtypeStruct((NCH, L, F), jnp.float32),
        ],
        scratch_shapes=[
            pltpu.VMEM((NCH, DS, CI), jnp.float32),
            pltpu.VMEM((NCH, DS, CI), jnp.float32),
            pltpu.VMEM((NCH, CI), jnp.float32),
        ],
    )(dt, u, z, bc, A_logT, D_skip, out_proj_W, out_proj_b,
      W_proj, b_proj, W_msg, b_msg)


def _sc_gnn_body(q_hbm, src_hbm, dst_hbm, zero_hbm, out_hbm,
                 src_v, dst_v, rows_v, agg_sh, sem):
    cid = lax.axis_index("c")
    sid = lax.axis_index("s")
    rpt = NROWS // 16
    nch = lax.select(cid == 0, CH0, CH1)
    base = lax.select(cid == 0, sid * CH0, 16 * CH0 + sid * CH1)
    pltpu.sync_copy(zero_hbm.at[pl.ds(sid * rpt, rpt)],
                    agg_sh.at[pl.ds(sid * rpt, rpt)])
    plsc.subcore_barrier()

    def body(i):
        ch = base + i
        pltpu.sync_copy(src_hbm.at[ch], src_v)
        pltpu.sync_copy(dst_hbm.at[ch], dst_v)
        pltpu.async_copy(q_hbm.at[src_v], rows_v, sem).wait()
        pltpu.sync_copy(rows_v, agg_sh.at[dst_v], add=True)

    pl.loop(0, nch)(body)
    plsc.subcore_barrier()
    pltpu.sync_copy(agg_sh.at[pl.ds(sid * rpt, rpt)],
                    out_hbm.at[cid, pl.ds(sid * rpt, rpt)])


def _sc_gnn(q, srcw, dstw, zeros_rows):
    kfn = pl.kernel(
        _sc_gnn_body,
        out_type=jax.ShapeDtypeStruct((2, NROWS, F), jnp.float32),
        mesh=plsc.VectorSubcoreMesh(core_axis_name="c", subcore_axis_name="s"),
        scratch_types=[
            pltpu.VMEM((KE,), jnp.int32),
            pltpu.VMEM((KE,), jnp.int32),
            pltpu.VMEM((KE, F), jnp.float32),
            pltpu.VMEM_SHARED((NROWS, F), jnp.float32),
            pltpu.SemaphoreType.DMA,
        ],
    )
    return kfn(q, srcw, dstw, zeros_rows)


def _post_body(p_ref, agg_ref, wupd_ref, bupd_ref, wout_ref, bout_ref, o_ref):
    p = p_ref[0]
    agg = agg_ref[0, 0] + agg_ref[1, 0]
    acc = jnp.dot(p, wupd_ref[:F], preferred_element_type=jnp.float32)
    acc = acc + jnp.dot(agg, wupd_ref[F:], preferred_element_type=jnp.float32)
    upd = _silu(acc + bupd_ref[...])
    o_ref[0] = jnp.dot(upd, wout_ref[...],
                       preferred_element_type=jnp.float32) + bout_ref[...]


def _post(p, agg, W_upd, b_upd, W_out, b_out):
    full = lambda shape: pl.BlockSpec(shape, lambda c: (0,) * len(shape))
    return pl.pallas_call(
        _post_body,
        grid=(NCH,),
        in_specs=[
            pl.BlockSpec((1, L, F), lambda c: (c, 0, 0)),
            pl.BlockSpec((2, 1, L, F), lambda c: (0, c, 0, 0)),
            full((2 * F, F)), full((1, F)), full((F, OUT)), full((1, OUT)),
        ],
        out_specs=pl.BlockSpec((1, L, OUT), lambda c: (c, 0, 0)),
        out_shape=jax.ShapeDtypeStruct((NCH, L, OUT), jnp.float32),
    )(p, agg, W_upd, b_upd, W_out, b_out)


def kernel(x, edge_index, W_emb, b_emb, in_proj_W, in_proj_b, conv_W, conv_b,
           x_proj_W, dt_proj_W, dt_proj_b, A_log, D_skip, out_proj_W,
           out_proj_b, W_proj, b_proj, W_msg, b_msg, W_upd, b_upd,
           W_out, b_out):
    dt, u, z, bc = _pre(
        x, W_emb, b_emb.reshape(1, C), in_proj_W, in_proj_b.reshape(1, 2 * CI),
        conv_W.T, conv_b.reshape(1, CI), x_proj_W, dt_proj_W,
        dt_proj_b.reshape(1, CI))

    p, q = _scan(dt.reshape(L, NCH, CI), u.reshape(L, NCH, CI),
                 z.reshape(L, NCH, CI), bc.reshape(L, NCH, 2 * DS),
                 A_log.T, D_skip.reshape(1, CI), out_proj_W,
                 out_proj_b.reshape(1, C), W_proj, b_proj.reshape(1, F),
                 W_msg, b_msg.reshape(1, F))
    q = q.reshape(N, F)

    npad = EPAD - E
    srcw = jnp.concatenate(
        [edge_index[0], jnp.zeros((npad,), jnp.int32)]).reshape(TOTCH, KE)
    dstw = jnp.concatenate(
        [edge_index[1], jnp.full((npad,), N, jnp.int32)]).reshape(TOTCH, KE)
    zeros_rows = jnp.zeros((NROWS, F), jnp.float32)
    agg = _sc_gnn(q, srcw, dstw, zeros_rows)[:, :N, :].reshape(2, NCH, L, F)

    out = _post(p, agg, W_upd, b_upd.reshape(1, F), W_out,
                b_out.reshape(1, OUT))
    return out.reshape(N, OUT)


# speedup vs baseline: 1.1454x; 1.1454x over previous
import functools

import jax
import jax.numpy as jnp
from jax import lax
from jax.experimental import pallas as pl
from jax.experimental.pallas import tpu as pltpu
from jax.experimental.pallas import tpu_sc as plsc

N = 10000
E = 320000
F = 128
C = 128
CI = 256
DS = 16
DC = 4
DTR = 8
OUT = 128

L = 125
NCH = 80

NW = 32
KE = 128
CH0 = 100
CH1 = 60
TOTCH = 16 * (CH0 + CH1)
EPAD = TOTCH * KE
NROWS = 10112


def _silu(a):
    return a * (1.0 / (1.0 + jnp.exp(-a)))


def _softplus(a):
    return jnp.log(1.0 + jnp.exp(-jnp.abs(a))) + jnp.maximum(a, 0.0)


def _pre_body(x_ref, wemb_ref, bemb_ref, winp_ref, binp_ref, convt_ref,
              bconv_ref, wxp_ref, wdt_ref, bdt_ref,
              dt_ref, u_ref, z_ref, bc_ref, halo_ref):
    c = pl.program_id(0)
    xblk = x_ref[0]
    h = jnp.dot(xblk, wemb_ref[...], preferred_element_type=jnp.float32)
    h = h + bemb_ref[...]
    xz = jnp.dot(h, winp_ref[...], preferred_element_type=jnp.float32)
    xz = xz + binp_ref[...]
    xm = xz[:, :CI]
    z = xz[:, CI:]

    @pl.when(c == 0)
    def _():
        halo_ref[...] = jnp.zeros_like(halo_ref)

    halo = halo_ref[...]
    xm_ext = jnp.concatenate([halo[5:8], xm], axis=0)
    halo_ref[5:8] = xm[L - 3:L]

    xc = (xm_ext[0:L] * convt_ref[0:1] + xm_ext[1:L + 1] * convt_ref[1:2]
          + xm_ext[2:L + 2] * convt_ref[2:3] + xm_ext[3:L + 3] * convt_ref[3:4])
    xm2 = _silu(xc + bconv_ref[...])
    x_dbl = jnp.dot(xm2, wxp_ref[...], preferred_element_type=jnp.float32)
    dt = _softplus(jnp.dot(x_dbl[:, :DTR], wdt_ref[...],
                           preferred_element_type=jnp.float32) + bdt_ref[...])

    dt_ref[...] = dt[:, None, None, :]
    u_ref[...] = xm2[:, None, None, :]
    z_ref[...] = z[:, None, None, :]
    bc_ref[...] = x_dbl[:, None, None, DTR:DTR + 2 * DS]


def _pre(x, W_emb, b_emb, in_proj_W, in_proj_b, conv_Wt, conv_b,
         x_proj_W, dt_proj_W, dt_proj_b):
    full = lambda shape: pl.BlockSpec(shape, lambda c: (0,) * len(shape))
    return pl.pallas_call(
        _pre_body,
        grid=(NCH,),
        in_specs=[
            pl.BlockSpec((1, L, F), lambda c: (c, 0, 0)),
            full((F, C)), full((1, C)), full((C, 2 * CI)), full((1, 2 * CI)),
            full((DC, CI)), full((1, CI)), full((CI, DTR + 2 * DS)),
            full((DTR, CI)), full((1, CI)),
        ],
        out_specs=[
            pl.BlockSpec((L, 1, 1, CI), lambda c: (0, c, 0, 0)),
            pl.BlockSpec((L, 1, 1, CI), lambda c: (0, c, 0, 0)),
            pl.BlockSpec((L, 1, 1, CI), lambda c: (0, c, 0, 0)),
            pl.BlockSpec((L, 1, 1, 2 * DS), lambda c: (0, c, 0, 0)),
        ],
        out_shape=[
            jax.ShapeDtypeStruct((L, NCH, 1, CI), jnp.float32),
            jax.ShapeDtypeStruct((L, NCH, 1, CI), jnp.float32),
            jax.ShapeDtypeStruct((L, NCH, 1, CI), jnp.float32),
            jax.ShapeDtypeStruct((L, NCH, 1, 2 * DS), jnp.float32),
        ],
        scratch_shapes=[pltpu.VMEM((8, CI), jnp.float32)],
    )(x.reshape(NCH, L, F), W_emb, b_emb, in_proj_W, in_proj_b, conv_Wt,
      conv_b, x_proj_W, dt_proj_W, dt_proj_b)


def _scan_body(dt_ref, u_ref, z_ref, bc_ref, alogt_ref, d_ref,
               wout_ref, bout_ref, wproj_ref, bproj_ref, wmsg_ref, bmsg_ref,
               p_ref, q_ref, s_ref, i_ref, g_ref, abc_ref):
    A_T = -jnp.exp(alogt_ref[...])
    abc_ref[...] = jnp.broadcast_to(A_T[:, None, :], (DS, NCH, CI))

    s_ref[...] = jnp.zeros_like(s_ref)

    def pass1(t, g):
        dt_t = dt_ref[t]
        u_t = u_ref[t]
        bt = bc_ref[t][:, :DS].transpose(1, 0)[:, :, None]
        dA = jnp.exp(dt_t[None] * abc_ref[...])
        s_ref[...] = s_ref[...] * dA + (dt_t * u_t)[None] * bt
        return g + dt_t

    g_ref[...] = lax.fori_loop(0, L, pass1, jnp.zeros((NCH, CI), jnp.float32))

    def comb(c, S):
        i_ref[:, c, :] = S
        decay = jnp.exp(g_ref[c][None, :] * A_T)
        return s_ref[:, c, :] + decay * S

    lax.fori_loop(0, NCH, comb, jnp.zeros((DS, CI), jnp.float32))

    def pass2(t, _):
        dt_t = dt_ref[t]
        u_t = u_ref[t]
        bc_t = bc_ref[t]
        bt = bc_t[:, :DS].transpose(1, 0)[:, :, None]
        ct = bc_t[:, DS:].transpose(1, 0)[:, :, None]
        dA = jnp.exp(dt_t[None] * abc_ref[...])
        s = i_ref[...] * dA + (dt_t * u_t)[None] * bt
        i_ref[...] = s
        y = jnp.sum(s * ct, axis=0) + d_ref[...] * u_t
        y = y * _silu(z_ref[t])
        h2 = jnp.dot(y, wout_ref[...], preferred_element_type=jnp.float32)
        h2 = h2 + bout_ref[...]
        p = jnp.dot(h2, wproj_ref[...], preferred_element_type=jnp.float32)
        p = p + bproj_ref[...]
        q = _silu(jnp.dot(p, wmsg_ref[...], preferred_element_type=jnp.float32)
                  + bmsg_ref[...])
        p_ref[:, t, :] = p
        q_ref[:, t, :] = q
        return 0

    lax.fori_loop(0, L, pass2, 0)


def _scan(dt, u, z, bc, A_logT, D_skip, out_proj_W, out_proj_b,
          W_proj, b_proj, W_msg, b_msg):
    return pl.pallas_call(
        _scan_body,
        out_shape=[
            jax.ShapeDtypeStruct((NCH, L, F), jnp.float32),
            jax.ShapeDtypeStruct((NCH, L, F), jnp.float32),
        ],
        scratch_shapes=[
            pltpu.VMEM((DS, NCH, CI), jnp.float32),
            pltpu.VMEM((DS, NCH, CI), jnp.float32),
            pltpu.VMEM((NCH, CI), jnp.float32),
            pltpu.VMEM((DS, NCH, CI), jnp.float32),
        ],
    )(dt, u, z, bc, A_logT, D_skip, out_proj_W, out_proj_b,
      W_proj, b_proj, W_msg, b_msg)


def _sc_gnn_body(q_hbm, src_hbm, dst_hbm, zero_hbm, out_hbm,
                 src_v, dst_v, rows_v, agg_sh, sem):
    cid = lax.axis_index("c")
    sid = lax.axis_index("s")
    rpt = NROWS // 16
    nch = lax.select(cid == 0, CH0, CH1)
    base = lax.select(cid == 0, sid * CH0, 16 * CH0 + sid * CH1)
    pltpu.sync_copy(zero_hbm.at[pl.ds(sid * rpt, rpt)],
                    agg_sh.at[pl.ds(sid * rpt, rpt)])
    plsc.subcore_barrier()

    def body(i):
        ch = base + i
        pltpu.sync_copy(src_hbm.at[ch], src_v)
        pltpu.sync_copy(dst_hbm.at[ch], dst_v)
        pltpu.async_copy(q_hbm.at[src_v], rows_v, sem).wait()
        pltpu.sync_copy(rows_v, agg_sh.at[dst_v], add=True)

    pl.loop(0, nch)(body)
    plsc.subcore_barrier()
    pltpu.sync_copy(agg_sh.at[pl.ds(sid * rpt, rpt)],
                    out_hbm.at[cid, pl.ds(sid * rpt, rpt)])


def _sc_gnn(q, srcw, dstw, zeros_rows):
    kfn = pl.kernel(
        _sc_gnn_body,
        out_type=jax.ShapeDtypeStruct((2, NROWS, F), jnp.float32),
        mesh=plsc.VectorSubcoreMesh(core_axis_name="c", subcore_axis_name="s"),
        scratch_types=[
            pltpu.VMEM((KE,), jnp.int32),
            pltpu.VMEM((KE,), jnp.int32),
            pltpu.VMEM((KE, F), jnp.float32),
            pltpu.VMEM_SHARED((NROWS, F), jnp.float32),
            pltpu.SemaphoreType.DMA,
        ],
    )
    return kfn(q, srcw, dstw, zeros_rows)


def _post_body(p_ref, agg_ref, wupd_ref, bupd_ref, wout_ref, bout_ref, o_ref):
    p = p_ref[0]
    agg = agg_ref[0, 0] + agg_ref[1, 0]
    acc = jnp.dot(p, wupd_ref[:F], preferred_element_type=jnp.float32)
    acc = acc + jnp.dot(agg, wupd_ref[F:], preferred_element_type=jnp.float32)
    upd = _silu(acc + bupd_ref[...])
    o_ref[0] = jnp.dot(upd, wout_ref[...],
                       preferred_element_type=jnp.float32) + bout_ref[...]


def _post(p, agg, W_upd, b_upd, W_out, b_out):
    full = lambda shape: pl.BlockSpec(shape, lambda c: (0,) * len(shape))
    return pl.pallas_call(
        _post_body,
        grid=(NCH,),
        in_specs=[
            pl.BlockSpec((1, L, F), lambda c: (c, 0, 0)),
            pl.BlockSpec((2, 1, L, F), lambda c: (0, c, 0, 0)),
            full((2 * F, F)), full((1, F)), full((F, OUT)), full((1, OUT)),
        ],
        out_specs=pl.BlockSpec((1, L, OUT), lambda c: (c, 0, 0)),
        out_shape=jax.ShapeDtypeStruct((NCH, L, OUT), jnp.float32),
    )(p, agg, W_upd, b_upd, W_out, b_out)


def kernel(x, edge_index, W_emb, b_emb, in_proj_W, in_proj_b, conv_W, conv_b,
           x_proj_W, dt_proj_W, dt_proj_b, A_log, D_skip, out_proj_W,
           out_proj_b, W_proj, b_proj, W_msg, b_msg, W_upd, b_upd,
           W_out, b_out):
    dt, u, z, bc = _pre(
        x, W_emb, b_emb.reshape(1, C), in_proj_W, in_proj_b.reshape(1, 2 * CI),
        conv_W.T, conv_b.reshape(1, CI), x_proj_W, dt_proj_W,
        dt_proj_b.reshape(1, CI))

    p, q = _scan(dt.reshape(L, NCH, CI), u.reshape(L, NCH, CI),
                 z.reshape(L, NCH, CI), bc.reshape(L, NCH, 2 * DS),
                 A_log.T, D_skip.reshape(1, CI), out_proj_W,
                 out_proj_b.reshape(1, C), W_proj, b_proj.reshape(1, F),
                 W_msg, b_msg.reshape(1, F))
    q = q.reshape(N, F)

    npad = EPAD - E
    srcw = jnp.concatenate(
        [edge_index[0], jnp.zeros((npad,), jnp.int32)]).reshape(TOTCH, KE)
    dstw = jnp.concatenate(
        [edge_index[1], jnp.full((npad,), N, jnp.int32)]).reshape(TOTCH, KE)
    zeros_rows = jnp.zeros((NROWS, F), jnp.float32)
    agg = _sc_gnn(q, srcw, dstw, zeros_rows)[:, :N, :].reshape(2, NCH, L, F)

    out = _post(p, agg, W_upd, b_upd.reshape(1, F), W_out,
                b_out.reshape(1, OUT))
    return out.reshape(N, OUT)


# speedup vs baseline: 1.3390x; 1.1691x over previous
import functools

import jax
import jax.numpy as jnp
from jax import lax
from jax.experimental import pallas as pl
from jax.experimental.pallas import tpu as pltpu
from jax.experimental.pallas import tpu_sc as plsc

N = 10000
E = 320000
F = 128
C = 128
CI = 256
DS = 16
DC = 4
DTR = 8
OUT = 128

L = 125
NCH = 80

NW = 32
KE = 128
CH0 = 80
CH1 = 80
TOTCH = 16 * (CH0 + CH1)
EPAD = TOTCH * KE
NROWS = 10112


def _silu(a):
    return a * (1.0 / (1.0 + jnp.exp(-a)))


def _softplus(a):
    return jnp.log(1.0 + jnp.exp(-jnp.abs(a))) + jnp.maximum(a, 0.0)


def _pre_body(x_ref, wemb_ref, bemb_ref, winp_ref, binp_ref, convt_ref,
              bconv_ref, wxp_ref, wdt_ref, bdt_ref,
              dt_ref, u_ref, z_ref, bc_ref, halo_ref):
    c = pl.program_id(0)
    xblk = x_ref[0]
    h = jnp.dot(xblk, wemb_ref[...], preferred_element_type=jnp.float32)
    h = h + bemb_ref[...]
    xz = jnp.dot(h, winp_ref[...], preferred_element_type=jnp.float32)
    xz = xz + binp_ref[...]
    xm = xz[:, :CI]
    z = xz[:, CI:]

    @pl.when(c == 0)
    def _():
        halo_ref[...] = jnp.zeros_like(halo_ref)

    halo = halo_ref[...]
    xm_ext = jnp.concatenate([halo[5:8], xm], axis=0)
    halo_ref[5:8] = xm[L - 3:L]

    xc = (xm_ext[0:L] * convt_ref[0:1] + xm_ext[1:L + 1] * convt_ref[1:2]
          + xm_ext[2:L + 2] * convt_ref[2:3] + xm_ext[3:L + 3] * convt_ref[3:4])
    xm2 = _silu(xc + bconv_ref[...])
    x_dbl = jnp.dot(xm2, wxp_ref[...], preferred_element_type=jnp.float32)
    dt = _softplus(jnp.dot(x_dbl[:, :DTR], wdt_ref[...],
                           preferred_element_type=jnp.float32) + bdt_ref[...])

    dt_ref[...] = dt[:, None, None, :]
    u_ref[...] = xm2[:, None, None, :]
    z_ref[...] = z[:, None, None, :]
    bc_ref[...] = x_dbl[:, None, None, DTR:DTR + 2 * DS]


def _pre(x, W_emb, b_emb, in_proj_W, in_proj_b, conv_Wt, conv_b,
         x_proj_W, dt_proj_W, dt_proj_b):
    full = lambda shape: pl.BlockSpec(shape, lambda c: (0,) * len(shape))
    return pl.pallas_call(
        _pre_body,
        grid=(NCH,),
        in_specs=[
            pl.BlockSpec((1, L, F), lambda c: (c, 0, 0)),
            full((F, C)), full((1, C)), full((C, 2 * CI)), full((1, 2 * CI)),
            full((DC, CI)), full((1, CI)), full((CI, DTR + 2 * DS)),
            full((DTR, CI)), full((1, CI)),
        ],
        out_specs=[
            pl.BlockSpec((L, 1, 1, CI), lambda c: (0, c, 0, 0)),
            pl.BlockSpec((L, 1, 1, CI), lambda c: (0, c, 0, 0)),
            pl.BlockSpec((L, 1, 1, CI), lambda c: (0, c, 0, 0)),
            pl.BlockSpec((L, 1, 1, 2 * DS), lambda c: (0, c, 0, 0)),
        ],
        out_shape=[
            jax.ShapeDtypeStruct((L, NCH, 1, CI), jnp.float32),
            jax.ShapeDtypeStruct((L, NCH, 1, CI), jnp.float32),
            jax.ShapeDtypeStruct((L, NCH, 1, CI), jnp.float32),
            jax.ShapeDtypeStruct((L, NCH, 1, 2 * DS), jnp.float32),
        ],
        scratch_shapes=[pltpu.VMEM((8, CI), jnp.float32)],
    )(x.reshape(NCH, L, F), W_emb, b_emb, in_proj_W, in_proj_b, conv_Wt,
      conv_b, x_proj_W, dt_proj_W, dt_proj_b)


def _scan_body(dt_ref, u_ref, z_ref, bc_ref, alogt_ref, d_ref,
               wout_ref, bout_ref, wproj_ref, bproj_ref, wmsg_ref, bmsg_ref,
               p_ref, q_ref, s_ref, i_ref, g_ref, abc_ref):
    A_T = -jnp.exp(alogt_ref[...])
    abc_ref[...] = jnp.broadcast_to(A_T[:, None, :], (DS, NCH, CI))

    s_ref[...] = jnp.zeros_like(s_ref)

    def pass1(t, g):
        dt_t = dt_ref[t]
        u_t = u_ref[t]
        bt = bc_ref[t][:, :DS].transpose(1, 0)[:, :, None]
        dA = jnp.exp(dt_t[None] * abc_ref[...])
        s_ref[...] = s_ref[...] * dA + (dt_t * u_t)[None] * bt
        return g + dt_t

    g_ref[...] = lax.fori_loop(0, L, pass1, jnp.zeros((NCH, CI), jnp.float32))

    def comb(c, S):
        i_ref[:, c, :] = S
        decay = jnp.exp(g_ref[c][None, :] * A_T)
        return s_ref[:, c, :] + decay * S

    lax.fori_loop(0, NCH, comb, jnp.zeros((DS, CI), jnp.float32))

    def pass2(t, _):
        dt_t = dt_ref[t]
        u_t = u_ref[t]
        bc_t = bc_ref[t]
        bt = bc_t[:, :DS].transpose(1, 0)[:, :, None]
        ct = bc_t[:, DS:].transpose(1, 0)[:, :, None]
        dA = jnp.exp(dt_t[None] * abc_ref[...])
        s = i_ref[...] * dA + (dt_t * u_t)[None] * bt
        i_ref[...] = s
        y = jnp.sum(s * ct, axis=0) + d_ref[...] * u_t
        y = y * _silu(z_ref[t])
        h2 = jnp.dot(y, wout_ref[...], preferred_element_type=jnp.float32)
        h2 = h2 + bout_ref[...]
        p = jnp.dot(h2, wproj_ref[...], preferred_element_type=jnp.float32)
        p = p + bproj_ref[...]
        q = _silu(jnp.dot(p, wmsg_ref[...], preferred_element_type=jnp.float32)
                  + bmsg_ref[...])
        p_ref[:, t, :] = p
        q_ref[:, t, :] = q
        return 0

    lax.fori_loop(0, L, pass2, 0)


def _scan(dt, u, z, bc, A_logT, D_skip, out_proj_W, out_proj_b,
          W_proj, b_proj, W_msg, b_msg):
    return pl.pallas_call(
        _scan_body,
        out_shape=[
            jax.ShapeDtypeStruct((NCH, L, F), jnp.float32),
            jax.ShapeDtypeStruct((NCH, L, F), jnp.float32),
        ],
        scratch_shapes=[
            pltpu.VMEM((DS, NCH, CI), jnp.float32),
            pltpu.VMEM((DS, NCH, CI), jnp.float32),
            pltpu.VMEM((NCH, CI), jnp.float32),
            pltpu.VMEM((DS, NCH, CI), jnp.float32),
        ],
    )(dt, u, z, bc, A_logT, D_skip, out_proj_W, out_proj_b,
      W_proj, b_proj, W_msg, b_msg)


def _sc_gnn_body(q_hbm, src_hbm, dst_hbm, zero_hbm, out_hbm,
                 src_v, dst_v, rows_v, agg_sh, sem):
    cid = lax.axis_index("c")
    sid = lax.axis_index("s")
    rpt = NROWS // 16
    nch = lax.select(cid == 0, CH0, CH1)
    base = lax.select(cid == 0, sid * CH0, 16 * CH0 + sid * CH1)
    pltpu.sync_copy(zero_hbm.at[pl.ds(sid * rpt, rpt)],
                    agg_sh.at[pl.ds(sid * rpt, rpt)])
    plsc.subcore_barrier()

    def body(i):
        ch = base + i
        pltpu.sync_copy(src_hbm.at[ch], src_v)
        pltpu.sync_copy(dst_hbm.at[ch], dst_v)
        pltpu.async_copy(q_hbm.at[src_v], rows_v, sem).wait()
        pltpu.sync_copy(rows_v, agg_sh.at[dst_v], add=True)

    pl.loop(0, nch)(body)
    plsc.subcore_barrier()
    pltpu.sync_copy(agg_sh.at[pl.ds(sid * rpt, rpt)],
                    out_hbm.at[cid, pl.ds(sid * rpt, rpt)])


def _sc_gnn(q, srcw, dstw, zeros_rows):
    kfn = pl.kernel(
        _sc_gnn_body,
        out_type=jax.ShapeDtypeStruct((2, NROWS, F), jnp.float32),
        mesh=plsc.VectorSubcoreMesh(core_axis_name="c", subcore_axis_name="s"),
        scratch_types=[
            pltpu.VMEM((KE,), jnp.int32),
            pltpu.VMEM((KE,), jnp.int32),
            pltpu.VMEM((KE, F), jnp.float32),
            pltpu.VMEM_SHARED((NROWS, F), jnp.float32),
            pltpu.SemaphoreType.DMA,
        ],
    )
    return kfn(q, srcw, dstw, zeros_rows)


def _post_body(p_ref, agg_ref, wupd_ref, bupd_ref, wout_ref, bout_ref, o_ref):
    p = p_ref[0]
    agg = agg_ref[0, 0] + agg_ref[1, 0]
    acc = jnp.dot(p, wupd_ref[:F], preferred_element_type=jnp.float32)
    acc = acc + jnp.dot(agg, wupd_ref[F:], preferred_element_type=jnp.float32)
    upd = _silu(acc + bupd_ref[...])
    o_ref[0] = jnp.dot(upd, wout_ref[...],
                       preferred_element_type=jnp.float32) + bout_ref[...]


def _post(p, agg, W_upd, b_upd, W_out, b_out):
    full = lambda shape: pl.BlockSpec(shape, lambda c: (0,) * len(shape))
    return pl.pallas_call(
        _post_body,
        grid=(NCH,),
        in_specs=[
            pl.BlockSpec((1, L, F), lambda c: (c, 0, 0)),
            pl.BlockSpec((2, 1, L, F), lambda c: (0, c, 0, 0)),
            full((2 * F, F)), full((1, F)), full((F, OUT)), full((1, OUT)),
        ],
        out_specs=pl.BlockSpec((1, L, OUT), lambda c: (c, 0, 0)),
        out_shape=jax.ShapeDtypeStruct((NCH, L, OUT), jnp.float32),
    )(p, agg, W_upd, b_upd, W_out, b_out)


def kernel(x, edge_index, W_emb, b_emb, in_proj_W, in_proj_b, conv_W, conv_b,
           x_proj_W, dt_proj_W, dt_proj_b, A_log, D_skip, out_proj_W,
           out_proj_b, W_proj, b_proj, W_msg, b_msg, W_upd, b_upd,
           W_out, b_out):
    dt, u, z, bc = _pre(
        x, W_emb, b_emb.reshape(1, C), in_proj_W, in_proj_b.reshape(1, 2 * CI),
        conv_W.T, conv_b.reshape(1, CI), x_proj_W, dt_proj_W,
        dt_proj_b.reshape(1, CI))

    p, q = _scan(dt.reshape(L, NCH, CI), u.reshape(L, NCH, CI),
                 z.reshape(L, NCH, CI), bc.reshape(L, NCH, 2 * DS),
                 A_log.T, D_skip.reshape(1, CI), out_proj_W,
                 out_proj_b.reshape(1, C), W_proj, b_proj.reshape(1, F),
                 W_msg, b_msg.reshape(1, F))
    q = q.reshape(N, F)

    kreal = E // TOTCH
    pad_d = (N + (jnp.arange(TOTCH, dtype=jnp.int32) % (NROWS - N)))[:, None]
    srcw = jnp.concatenate(
        [edge_index[0].reshape(TOTCH, kreal),
         jnp.zeros((TOTCH, KE - kreal), jnp.int32)], axis=1)
    dstw = jnp.concatenate(
        [edge_index[1].reshape(TOTCH, kreal),
         jnp.broadcast_to(pad_d, (TOTCH, KE - kreal))], axis=1)
    zeros_rows = jnp.zeros((NROWS, F), jnp.float32)
    agg = _sc_gnn(q, srcw, dstw, zeros_rows)[:, :N, :].reshape(2, NCH, L, F)

    out = _post(p, agg, W_upd, b_upd.reshape(1, F), W_out,
                b_out.reshape(1, OUT))
    return out.reshape(N, OUT)


# speedup vs baseline: 1.3395x; 1.0004x over previous
import functools

import jax
import jax.numpy as jnp
from jax import lax
from jax.experimental import pallas as pl
from jax.experimental.pallas import tpu as pltpu
from jax.experimental.pallas import tpu_sc as plsc

N = 10000
E = 320000
F = 128
C = 128
CI = 256
DS = 16
DC = 4
DTR = 8
OUT = 128

L = 125
NCH = 80

NW = 32
KE = 128
CHW = 80
TOTCH = NW * CHW
EPAD = TOTCH * KE
NROWS = 10112


def _silu(a):
    return a * (1.0 / (1.0 + jnp.exp(-a)))


def _softplus(a):
    return jnp.log(1.0 + jnp.exp(-jnp.abs(a))) + jnp.maximum(a, 0.0)


def _pre_body(x_ref, wemb_ref, bemb_ref, winp_ref, binp_ref, convt_ref,
              bconv_ref, wxp_ref, wdt_ref, bdt_ref,
              dt_ref, u_ref, z_ref, bc_ref, halo_ref):
    c = pl.program_id(0)
    xblk = x_ref[0]
    h = jnp.dot(xblk, wemb_ref[...], preferred_element_type=jnp.float32)
    h = h + bemb_ref[...]
    xz = jnp.dot(h, winp_ref[...], preferred_element_type=jnp.float32)
    xz = xz + binp_ref[...]
    xm = xz[:, :CI]
    z = xz[:, CI:]

    @pl.when(c == 0)
    def _():
        halo_ref[...] = jnp.zeros_like(halo_ref)

    halo = halo_ref[...]
    xm_ext = jnp.concatenate([halo[5:8], xm], axis=0)
    halo_ref[5:8] = xm[L - 3:L]

    xc = (xm_ext[0:L] * convt_ref[0:1] + xm_ext[1:L + 1] * convt_ref[1:2]
          + xm_ext[2:L + 2] * convt_ref[2:3] + xm_ext[3:L + 3] * convt_ref[3:4])
    xm2 = _silu(xc + bconv_ref[...])
    x_dbl = jnp.dot(xm2, wxp_ref[...], preferred_element_type=jnp.float32)
    dt = _softplus(jnp.dot(x_dbl[:, :DTR], wdt_ref[...],
                           preferred_element_type=jnp.float32) + bdt_ref[...])

    dt_ref[...] = dt[:, None, None, :]
    u_ref[...] = xm2[:, None, None, :]
    z_ref[...] = z[:, None, None, :]
    bc_ref[...] = x_dbl[:, None, None, DTR:DTR + 2 * DS]


def _pre(x, W_emb, b_emb, in_proj_W, in_proj_b, conv_Wt, conv_b,
         x_proj_W, dt_proj_W, dt_proj_b):
    full = lambda shape: pl.BlockSpec(shape, lambda c: (0,) * len(shape))
    return pl.pallas_call(
        _pre_body,
        grid=(NCH,),
        in_specs=[
            pl.BlockSpec((1, L, F), lambda c: (c, 0, 0)),
            full((F, C)), full((1, C)), full((C, 2 * CI)), full((1, 2 * CI)),
            full((DC, CI)), full((1, CI)), full((CI, DTR + 2 * DS)),
            full((DTR, CI)), full((1, CI)),
        ],
        out_specs=[
            pl.BlockSpec((L, 1, 1, CI), lambda c: (0, c, 0, 0)),
            pl.BlockSpec((L, 1, 1, CI), lambda c: (0, c, 0, 0)),
            pl.BlockSpec((L, 1, 1, CI), lambda c: (0, c, 0, 0)),
            pl.BlockSpec((L, 1, 1, 2 * DS), lambda c: (0, c, 0, 0)),
        ],
        out_shape=[
            jax.ShapeDtypeStruct((L, NCH, 1, CI), jnp.float32),
            jax.ShapeDtypeStruct((L, NCH, 1, CI), jnp.float32),
            jax.ShapeDtypeStruct((L, NCH, 1, CI), jnp.float32),
            jax.ShapeDtypeStruct((L, NCH, 1, 2 * DS), jnp.float32),
        ],
        scratch_shapes=[pltpu.VMEM((8, CI), jnp.float32)],
    )(x.reshape(NCH, L, F), W_emb, b_emb, in_proj_W, in_proj_b, conv_Wt,
      conv_b, x_proj_W, dt_proj_W, dt_proj_b)


def _scan_body(dt_ref, u_ref, z_ref, bc_ref, alogt_ref, d_ref,
               wout_ref, bout_ref, wproj_ref, bproj_ref, wmsg_ref, bmsg_ref,
               p_ref, q_ref, s_ref, i_ref, g_ref, abc_ref):
    A_T = -jnp.exp(alogt_ref[...])
    abc_ref[...] = jnp.broadcast_to(A_T[:, None, :], (DS, NCH, CI))

    s_ref[...] = jnp.zeros_like(s_ref)

    def pass1(t, g):
        dt_t = dt_ref[t]
        u_t = u_ref[t]
        bt = bc_ref[t][:, :DS].transpose(1, 0)[:, :, None]
        dA = jnp.exp(dt_t[None] * abc_ref[...])
        s_ref[...] = s_ref[...] * dA + (dt_t * u_t)[None] * bt
        return g + dt_t

    g_ref[...] = lax.fori_loop(0, L, pass1, jnp.zeros((NCH, CI), jnp.float32))

    def comb(c, S):
        i_ref[:, c, :] = S
        decay = jnp.exp(g_ref[c][None, :] * A_T)
        return s_ref[:, c, :] + decay * S

    lax.fori_loop(0, NCH, comb, jnp.zeros((DS, CI), jnp.float32))

    def pass2(t, _):
        dt_t = dt_ref[t]
        u_t = u_ref[t]
        bc_t = bc_ref[t]
        bt = bc_t[:, :DS].transpose(1, 0)[:, :, None]
        ct = bc_t[:, DS:].transpose(1, 0)[:, :, None]
        dA = jnp.exp(dt_t[None] * abc_ref[...])
        s = i_ref[...] * dA + (dt_t * u_t)[None] * bt
        i_ref[...] = s
        y = jnp.sum(s * ct, axis=0) + d_ref[...] * u_t
        y = y * _silu(z_ref[t])
        h2 = jnp.dot(y, wout_ref[...], preferred_element_type=jnp.float32)
        h2 = h2 + bout_ref[...]
        p = jnp.dot(h2, wproj_ref[...], preferred_element_type=jnp.float32)
        p = p + bproj_ref[...]
        q = _silu(jnp.dot(p, wmsg_ref[...], preferred_element_type=jnp.float32)
                  + bmsg_ref[...])
        p_ref[:, t, :] = p
        q_ref[:, t, :] = q
        return 0

    lax.fori_loop(0, L, pass2, 0)


def _scan(dt, u, z, bc, A_logT, D_skip, out_proj_W, out_proj_b,
          W_proj, b_proj, W_msg, b_msg):
    return pl.pallas_call(
        _scan_body,
        out_shape=[
            jax.ShapeDtypeStruct((NCH, L, F), jnp.float32),
            jax.ShapeDtypeStruct((NCH, L, F), jnp.float32),
        ],
        scratch_shapes=[
            pltpu.VMEM((DS, NCH, CI), jnp.float32),
            pltpu.VMEM((DS, NCH, CI), jnp.float32),
            pltpu.VMEM((NCH, CI), jnp.float32),
            pltpu.VMEM((DS, NCH, CI), jnp.float32),
        ],
    )(dt, u, z, bc, A_logT, D_skip, out_proj_W, out_proj_b,
      W_proj, b_proj, W_msg, b_msg)


def _sc_gnn_body(q_hbm, src_hbm, dst_hbm, zero_hbm, out_hbm,
                 si0, si1, dsts_v, rows0, rows1, agg_sh,
                 sgi0, sgi1, sg0, sg1, ss0, ss1):
    cid = lax.axis_index("c")
    sid = lax.axis_index("s")
    wid = cid * 16 + sid
    base = wid * CHW
    rpt = NROWS // 16
    pltpu.async_copy(src_hbm.at[base], si0, sgi0)
    pltpu.async_copy(src_hbm.at[base + 1], si1, sgi1)
    pltpu.sync_copy(zero_hbm.at[pl.ds(sid * rpt, rpt)],
                    agg_sh.at[pl.ds(sid * rpt, rpt)])
    pltpu.sync_copy(dst_hbm.at[pl.ds(base, CHW)], dsts_v)
    plsc.subcore_barrier()

    si = (si0, si1)
    rows = (rows0, rows1)
    sgi = (sgi0, sgi1)
    sg = (sg0, sg1)
    ss = (ss0, ss1)
    pltpu.make_async_copy(src_hbm.at[base], si0, sgi0).wait()
    pltpu.async_copy(q_hbm.at[si0], rows0, sg0)

    def body(bch):
        for b in range(2):
            ch = bch + b
            o = 1 - b
            pltpu.make_async_copy(q_hbm.at[si[b]], rows[b], sg[b]).wait()

            @pl.when(ch + 2 < CHW)
            def _():
                pltpu.async_copy(src_hbm.at[base + ch + 2], si[b], sgi[b])

            pltpu.async_copy(rows[b], agg_sh.at[dsts_v.at[ch]], ss[b],
                             add=True)

            @pl.when(ch + 1 < CHW)
            def _():
                @pl.when(ch >= 1)
                def _():
                    pltpu.make_async_copy(rows[o], agg_sh.at[dsts_v.at[ch]],
                                          ss[o]).wait()
                pltpu.make_async_copy(src_hbm.at[base + ch + 1], si[o],
                                      sgi[o]).wait()
                pltpu.async_copy(q_hbm.at[si[o]], rows[o], sg[o])

    pl.loop(0, CHW, step=2)(body)
    pltpu.make_async_copy(rows[0], agg_sh.at[dsts_v.at[0]], ss[0]).wait()
    pltpu.make_async_copy(rows[1], agg_sh.at[dsts_v.at[0]], ss[1]).wait()
    plsc.subcore_barrier()
    pltpu.sync_copy(agg_sh.at[pl.ds(sid * rpt, rpt)],
                    out_hbm.at[cid, pl.ds(sid * rpt, rpt)])


def _sc_gnn(q, srcw, dstw, zeros_rows):
    kfn = pl.kernel(
        _sc_gnn_body,
        out_type=jax.ShapeDtypeStruct((2, NROWS, F), jnp.float32),
        mesh=plsc.VectorSubcoreMesh(core_axis_name="c", subcore_axis_name="s"),
        scratch_types=[
            pltpu.VMEM((KE,), jnp.int32),
            pltpu.VMEM((KE,), jnp.int32),
            pltpu.VMEM((CHW, KE), jnp.int32),
            pltpu.VMEM((KE, F), jnp.float32),
            pltpu.VMEM((KE, F), jnp.float32),
            pltpu.VMEM_SHARED((NROWS, F), jnp.float32),
            pltpu.SemaphoreType.DMA,
            pltpu.SemaphoreType.DMA,
            pltpu.SemaphoreType.DMA,
            pltpu.SemaphoreType.DMA,
            pltpu.SemaphoreType.DMA,
            pltpu.SemaphoreType.DMA,
        ],
    )
    return kfn(q, srcw, dstw, zeros_rows)


def _post_body(p_ref, agg_ref, wupd_ref, bupd_ref, wout_ref, bout_ref, o_ref):
    p = p_ref[0]
    agg = agg_ref[0, 0] + agg_ref[1, 0]
    acc = jnp.dot(p, wupd_ref[:F], preferred_element_type=jnp.float32)
    acc = acc + jnp.dot(agg, wupd_ref[F:], preferred_element_type=jnp.float32)
    upd = _silu(acc + bupd_ref[...])
    o_ref[0] = jnp.dot(upd, wout_ref[...],
                       preferred_element_type=jnp.float32) + bout_ref[...]


def _post(p, agg, W_upd, b_upd, W_out, b_out):
    full = lambda shape: pl.BlockSpec(shape, lambda c: (0,) * len(shape))
    return pl.pallas_call(
        _post_body,
        grid=(NCH,),
        in_specs=[
            pl.BlockSpec((1, L, F), lambda c: (c, 0, 0)),
            pl.BlockSpec((2, 1, L, F), lambda c: (0, c, 0, 0)),
            full((2 * F, F)), full((1, F)), full((F, OUT)), full((1, OUT)),
        ],
        out_specs=pl.BlockSpec((1, L, OUT), lambda c: (c, 0, 0)),
        out_shape=jax.ShapeDtypeStruct((NCH, L, OUT), jnp.float32),
    )(p, agg, W_upd, b_upd, W_out, b_out)


def kernel(x, edge_index, W_emb, b_emb, in_proj_W, in_proj_b, conv_W, conv_b,
           x_proj_W, dt_proj_W, dt_proj_b, A_log, D_skip, out_proj_W,
           out_proj_b, W_proj, b_proj, W_msg, b_msg, W_upd, b_upd,
           W_out, b_out):
    dt, u, z, bc = _pre(
        x, W_emb, b_emb.reshape(1, C), in_proj_W, in_proj_b.reshape(1, 2 * CI),
        conv_W.T, conv_b.reshape(1, CI), x_proj_W, dt_proj_W,
        dt_proj_b.reshape(1, CI))

    p, q = _scan(dt.reshape(L, NCH, CI), u.reshape(L, NCH, CI),
                 z.reshape(L, NCH, CI), bc.reshape(L, NCH, 2 * DS),
                 A_log.T, D_skip.reshape(1, CI), out_proj_W,
                 out_proj_b.reshape(1, C), W_proj, b_proj.reshape(1, F),
                 W_msg, b_msg.reshape(1, F))
    q = q.reshape(N, F)

    kreal = E // TOTCH
    pad_d = (N + (jnp.arange(TOTCH, dtype=jnp.int32) % (NROWS - N)))[:, None]
    srcw = jnp.concatenate(
        [edge_index[0].reshape(TOTCH, kreal),
         jnp.zeros((TOTCH, KE - kreal), jnp.int32)], axis=1)
    dstw = jnp.concatenate(
        [edge_index[1].reshape(TOTCH, kreal),
         jnp.broadcast_to(pad_d, (TOTCH, KE - kreal))], axis=1)
    zeros_rows = jnp.zeros((NROWS, F), jnp.float32)
    agg = _sc_gnn(q, srcw, dstw, zeros_rows)[:, :N, :].reshape(2, NCH, L, F)

    out = _post(p, agg, W_upd, b_upd.reshape(1, F), W_out,
                b_out.reshape(1, OUT))
    return out.reshape(N, OUT)
